# Initial kernel scaffold; baseline (speedup 1.0000x reference)
import jax
import jax.numpy as jnp
from jax.experimental import pallas as pl


def kernel(self_vectors, neighbor_vectors, masks, W, b):
    raise NotImplementedError("write your pallas kernel here")


# speedup vs baseline: 1.1679x; 1.1679x over previous
import jax
import jax.numpy as jnp
from jax.experimental import pallas as pl
from jax.experimental.pallas import tpu as pltpu

_B = 4096
_D = 128
_K = 2
_N = 32
_BB = 256


def _body(nbr_ref, m_ref, sv_ref, wt_ref, b_ref, out_ref):
    nbr = nbr_ref[...]
    m = m_ref[...]
    e = jnp.sum(nbr * m[..., None], axis=2)
    scale = jnp.float32(1.0 / _N)
    x0 = sv_ref[...]
    e0 = e[:, 0, :] * scale
    e1 = e[:, 1, :] * scale
    acc = jnp.dot(x0, wt_ref[0:_D, :], preferred_element_type=jnp.float32)
    acc += jnp.dot(e0, wt_ref[_D:2 * _D, :], preferred_element_type=jnp.float32)
    acc += jnp.dot(e1, wt_ref[2 * _D:3 * _D, :], preferred_element_type=jnp.float32)
    out_ref[...] = acc + b_ref[...]


def kernel(self_vectors, neighbor_vectors, masks, W, b):
    nbr = neighbor_vectors.reshape(_B, _K, _N, _D)
    m = masks.reshape(_B, _K, _N)
    sv = self_vectors.reshape(_B, _D)
    wt = W.T
    bb = b.reshape(1, _D)

    grid = (_B // _BB,)
    out = pl.pallas_call(
        _body,
        grid=grid,
        in_specs=[
            pl.BlockSpec((_BB, _K, _N, _D), lambda i: (i, 0, 0, 0)),
            pl.BlockSpec((_BB, _K, _N), lambda i: (i, 0, 0)),
            pl.BlockSpec((_BB, _D), lambda i: (i, 0)),
            pl.BlockSpec((3 * _D, _D), lambda i: (0, 0)),
            pl.BlockSpec((1, _D), lambda i: (0, 0)),
        ],
        out_specs=pl.BlockSpec((_BB, _D), lambda i: (i, 0)),
        out_shape=jax.ShapeDtypeStruct((_B, _D), jnp.float32),
        compiler_params=pltpu.CompilerParams(
            dimension_semantics=("arbitrary",),
        ),
    )(nbr, m, sv, wt, bb)
    return out.reshape(_B, 1, _D)
